# baseline (speedup 1.0000x reference)
import numpy as np
import jax
import jax.numpy as jnp
from jax.experimental import pallas as pl
from jax.experimental.pallas import tpu as pltpu

BI = 128


def _build_t1():
    t1 = np.zeros((224, 96, 25), np.float32)
    for pi in range(8):
        for c in range(28):
            for dy in range(2):
                for dx in range(2):
                    for mpar in range(2):
                        for u in range(12):
                            ky = pi - 2 * mpar - dy
                            kx = c - 2 * u - dx
                            if 0 <= ky < 5 and 0 <= kx < 5:
                                t1[pi * 28 + c,
                                   ((dy * 2 + dx) * 2 + mpar) * 12 + u,
                                   ky * 5 + kx] = 1.0
    return t1


def _t2_tap_table():
    tab = -np.ones((72, 16), np.int32)
    for r6 in range(6):
        for j in range(12):
            for dy in range(2):
                for dx in range(2):
                    for u2 in range(4):
                        ky = r6 - dy
                        kx = j - 2 * u2 - dx
                        if 0 <= ky < 5 and 0 <= kx < 5:
                            tab[r6 * 12 + j, (dy * 2 + dx) * 4 + u2] = ky * 5 + kx
    return tab


_T2TAB = _t2_tap_table()


def _w2band_kernel(w2_ref, wl1_ref, out_ref, wl1r_ref):
    zero = jnp.zeros((64, 50), jnp.float32)
    for a in range(72):
        pieces = [w2_ref[int(t)] if t >= 0 else zero for t in _T2TAB[a]]
        out_ref[a * 64:(a + 1) * 64, :] = (
            jnp.concatenate(pieces, axis=1).astype(jnp.bfloat16))
    for m2 in range(4):
        wl1r_ref[:, m2 * 100:(m2 + 1) * 100] = (
            wl1_ref[m2 * 200:(m2 + 1) * 200, :].astype(jnp.bfloat16))


def _build_bands(w2, wl1):
    return pl.pallas_call(
        _w2band_kernel,
        out_shape=(jax.ShapeDtypeStruct((4608, 800), jnp.bfloat16),
                   jax.ShapeDtypeStruct((200, 400), jnp.bfloat16)),
    )(w2, wl1)


_T1 = _build_t1()


def _digit5_kernel(x_ref, w1b_ref, w2b_ref, b2t_ref,
                   wl1_ref, bl1_ref, wl2_ref, bl2_ref, wl3_ref, bl3_ref,
                   out_ref):
    f32 = jnp.float32
    bf16 = jnp.bfloat16
    x = x_ref[...].astype(bf16)

    x4 = x.reshape(BI, 7, 4, 28)
    ones = jnp.ones((BI, 6, 2), bf16)
    p1 = jnp.concatenate(
        [x4[:, d:d + 6, s, :] for d in range(2) for s in range(4)] + [ones],
        axis=2)
    p1 = jnp.concatenate([p1, jnp.zeros((BI, 2, 226), bf16)], axis=1)
    p1 = p1.reshape(BI * 8, 226)
    h1a = jnp.dot(p1, w1b_ref[:, 0:3072], preferred_element_type=f32)
    pa = jnp.maximum(h1a[:, 0:1536], h1a[:, 1536:3072])
    h1b = jnp.dot(p1, w1b_ref[:, 3072:6144], preferred_element_type=f32)
    pb = jnp.maximum(h1b[:, 0:1536], h1b[:, 1536:3072])
    pooled1 = jnp.maximum(jnp.maximum(pa, pb), 0.0).astype(bf16)

    xp2 = pooled1.reshape(BI, 8, 1536)
    p2 = jnp.concatenate([xp2[:, q:q + 4, :] for q in range(3)], axis=2)
    p2 = p2.reshape(BI * 4, 4608)
    h2 = jnp.dot(p2, w2b_ref[...], preferred_element_type=f32) + b2t_ref[...]
    h2 = jnp.maximum(jnp.maximum(h2[:, 0:200], h2[:, 200:400]),
                     jnp.maximum(h2[:, 400:600], h2[:, 600:800]))
    feats = jnp.maximum(h2, 0.0).astype(bf16)

    pfc = jnp.dot(feats, wl1_ref[...], preferred_element_type=f32)
    pfc = pfc.reshape(BI, 4, 400)
    h = (pfc[:, 0, 0:100] + pfc[:, 1, 100:200] + pfc[:, 2, 200:300]
         + pfc[:, 3, 300:400] + bl1_ref[...])
    h = jnp.maximum(h, 0.0).astype(bf16)
    h = jnp.dot(h, wl2_ref[...], preferred_element_type=f32) + bl2_ref[...]
    h = jnp.maximum(h, 0.0).astype(bf16)
    z = jnp.dot(h, wl3_ref[...], preferred_element_type=f32) + bl3_ref[...]
    m = jnp.max(z, axis=-1, keepdims=True)
    lse = jnp.log(jnp.sum(jnp.exp(z - m), axis=-1, keepdims=True)) + m
    out_ref[...] = z - lse


def kernel(x, w1, b1, w2, b2, wl1, bl1, wl2, bl2, wl3, bl3, p1, s2, p2):
    B = x.shape[0]
    xp = x.reshape(B, 28, 28)

    w1eff = jnp.sum(w1, axis=1)
    w1band = jnp.einsum("kgt,to->kgo", _T1, w1eff).reshape(224, 6144)
    b1t = jnp.tile(b1, (1, 96))
    b1hi = b1t.astype(jnp.bfloat16).astype(jnp.float32)
    w1band = jnp.concatenate([w1band, b1hi, b1t - b1hi], axis=0)
    w1band = w1band.astype(jnp.bfloat16)
    w2band, wl1r = _build_bands(w2, wl1)
    b2t = jnp.tile(b2, (1, 16))
    wl2 = wl2.astype(jnp.bfloat16)
    wl3 = wl3.astype(jnp.bfloat16)

    in_specs = [
        pl.BlockSpec((BI, 28, 28), lambda b: (b, 0, 0)),
        pl.BlockSpec((226, 6144), lambda b: (0, 0)),
        pl.BlockSpec((4608, 800), lambda b: (0, 0)),
        pl.BlockSpec((1, 800), lambda b: (0, 0)),
        pl.BlockSpec((200, 400), lambda b: (0, 0)),
        pl.BlockSpec((1, 100), lambda b: (0, 0)),
        pl.BlockSpec((100, 100), lambda b: (0, 0)),
        pl.BlockSpec((1, 100), lambda b: (0, 0)),
        pl.BlockSpec((100, 10), lambda b: (0, 0)),
        pl.BlockSpec((1, 10), lambda b: (0, 0)),
    ]
    return pl.pallas_call(
        _digit5_kernel,
        out_shape=jax.ShapeDtypeStruct((B, 10), jnp.float32),
        grid=(B // BI,),
        in_specs=in_specs,
        out_specs=pl.BlockSpec((BI, 10), lambda b: (b, 0)),
        compiler_params=pltpu.CompilerParams(
            dimension_semantics=("parallel",),
            vmem_limit_bytes=60 * 1024 * 1024,
        ),
    )(xp, w1band, w2band, b2t, wl1r, bl1, wl2, bl2, wl3, bl3)


# speedup vs baseline: 1.0623x; 1.0623x over previous
import numpy as np
import jax
import jax.numpy as jnp
from jax.experimental import pallas as pl
from jax.experimental.pallas import tpu as pltpu

BI = 128


def _build_t1():
    t1 = np.zeros((224, 96, 25), np.float32)
    for pi in range(8):
        for c in range(28):
            for dy in range(2):
                for dx in range(2):
                    for mpar in range(2):
                        for u in range(12):
                            ky = pi - 2 * mpar - dy
                            kx = c - 2 * u - dx
                            if 0 <= ky < 5 and 0 <= kx < 5:
                                t1[pi * 28 + c,
                                   ((dy * 2 + dx) * 2 + mpar) * 12 + u,
                                   ky * 5 + kx] = 1.0
    return t1


def _t2_tap_table():
    tab = -np.ones((72, 16), np.int32)
    for r6 in range(6):
        for j in range(12):
            for dy in range(2):
                for dx in range(2):
                    for u2 in range(4):
                        ky = r6 - dy
                        kx = j - 2 * u2 - dx
                        if 0 <= ky < 5 and 0 <= kx < 5:
                            tab[r6 * 12 + j, (dy * 2 + dx) * 4 + u2] = ky * 5 + kx
    return tab


_T2TAB = _t2_tap_table()


def _w2band_kernel(w2_ref, wl1_ref, out_ref, wl1r_ref):
    zero = jnp.zeros((64, 50), jnp.float32)
    for a in range(72):
        pieces = [w2_ref[int(t)] if t >= 0 else zero for t in _T2TAB[a]]
        out_ref[a * 64:(a + 1) * 64, :] = (
            jnp.concatenate(pieces, axis=1).astype(jnp.bfloat16))
    for m2 in range(4):
        wl1r_ref[:, m2 * 100:(m2 + 1) * 100] = (
            wl1_ref[m2 * 200:(m2 + 1) * 200, :].astype(jnp.bfloat16))


def _build_bands(w2, wl1):
    return pl.pallas_call(
        _w2band_kernel,
        out_shape=(jax.ShapeDtypeStruct((4608, 800), jnp.bfloat16),
                   jax.ShapeDtypeStruct((200, 400), jnp.bfloat16)),
    )(w2, wl1)


_T1 = _build_t1()


def _digit5_kernel(x_ref, w1b_ref, w2b_ref, b2t_ref,
                   wl1_ref, bl1_ref, wl2_ref, bl2_ref, wl3_ref, bl3_ref,
                   out_ref):
    f32 = jnp.float32
    bf16 = jnp.bfloat16
    x = x_ref[...].astype(bf16)

    x4 = x.reshape(BI, 7, 4, 28)
    ones = jnp.ones((BI, 6, 2), bf16)
    p1 = jnp.concatenate(
        [x4[:, d:d + 6, s, :] for d in range(2) for s in range(4)] + [ones],
        axis=2)
    p1 = p1.reshape(BI * 6, 226)
    h1 = jnp.dot(p1, w1b_ref[...], preferred_element_type=f32)
    h1 = jnp.maximum(jnp.maximum(h1[:, 0:1536], h1[:, 1536:3072]),
                     jnp.maximum(h1[:, 3072:4608], h1[:, 4608:6144]))
    pooled1 = jnp.maximum(h1, 0.0).astype(bf16)

    xp2 = pooled1.reshape(BI, 6, 1536)
    p2 = jnp.concatenate([xp2[:, q:q + 4, :] for q in range(3)], axis=2)
    p2 = p2.reshape(BI * 4, 4608)
    h2 = jnp.dot(p2, w2b_ref[...], preferred_element_type=f32) + b2t_ref[...]
    h2 = jnp.maximum(jnp.maximum(h2[:, 0:200], h2[:, 200:400]),
                     jnp.maximum(h2[:, 400:600], h2[:, 600:800]))
    feats = jnp.maximum(h2, 0.0).astype(bf16)

    pfc = jnp.dot(feats, wl1_ref[...], preferred_element_type=f32)
    pfc = pfc.reshape(BI, 4, 400)
    h = (pfc[:, 0, 0:100] + pfc[:, 1, 100:200] + pfc[:, 2, 200:300]
         + pfc[:, 3, 300:400] + bl1_ref[...])
    h = jnp.maximum(h, 0.0).astype(bf16)
    h = jnp.dot(h, wl2_ref[...], preferred_element_type=f32) + bl2_ref[...]
    h = jnp.maximum(h, 0.0).astype(bf16)
    z = jnp.dot(h, wl3_ref[...], preferred_element_type=f32) + bl3_ref[...]
    m = jnp.max(z, axis=-1, keepdims=True)
    lse = jnp.log(jnp.sum(jnp.exp(z - m), axis=-1, keepdims=True)) + m
    out_ref[...] = z - lse


def kernel(x, w1, b1, w2, b2, wl1, bl1, wl2, bl2, wl3, bl3, p1, s2, p2):
    B = x.shape[0]
    xp = x.reshape(B, 28, 28)

    w1eff = jnp.sum(w1, axis=1)
    w1band = jnp.einsum("kgt,to->kgo", _T1, w1eff).reshape(224, 6144)
    b1t = jnp.tile(b1, (1, 96))
    b1hi = b1t.astype(jnp.bfloat16).astype(jnp.float32)
    w1band = jnp.concatenate([w1band, b1hi, b1t - b1hi], axis=0)
    w1band = w1band.astype(jnp.bfloat16)
    w2band, wl1r = _build_bands(w2, wl1)
    b2t = jnp.tile(b2, (1, 16))
    wl2 = wl2.astype(jnp.bfloat16)
    wl3 = wl3.astype(jnp.bfloat16)

    in_specs = [
        pl.BlockSpec((BI, 28, 28), lambda b: (b, 0, 0)),
        pl.BlockSpec((226, 6144), lambda b: (0, 0)),
        pl.BlockSpec((4608, 800), lambda b: (0, 0)),
        pl.BlockSpec((1, 800), lambda b: (0, 0)),
        pl.BlockSpec((200, 400), lambda b: (0, 0)),
        pl.BlockSpec((1, 100), lambda b: (0, 0)),
        pl.BlockSpec((100, 100), lambda b: (0, 0)),
        pl.BlockSpec((1, 100), lambda b: (0, 0)),
        pl.BlockSpec((100, 10), lambda b: (0, 0)),
        pl.BlockSpec((1, 10), lambda b: (0, 0)),
    ]
    return pl.pallas_call(
        _digit5_kernel,
        out_shape=jax.ShapeDtypeStruct((B, 10), jnp.float32),
        grid=(B // BI,),
        in_specs=in_specs,
        out_specs=pl.BlockSpec((BI, 10), lambda b: (b, 0)),
        compiler_params=pltpu.CompilerParams(
            dimension_semantics=("parallel",),
            vmem_limit_bytes=60 * 1024 * 1024,
        ),
    )(xp, w1band, w2band, b2t, wl1r, bl1, wl2, bl2, wl3, bl3)


# speedup vs baseline: 1.1275x; 1.0615x over previous
import numpy as np
import jax
import jax.numpy as jnp
from jax.experimental import pallas as pl
from jax.experimental.pallas import tpu as pltpu

BI = 128


def _build_t1():
    t1 = np.zeros((224, 96, 25), np.float32)
    for pi in range(8):
        for c in range(28):
            for dy in range(2):
                for dx in range(2):
                    for mpar in range(2):
                        for u in range(12):
                            ky = pi - 2 * mpar - dy
                            kx = c - 2 * u - dx
                            if 0 <= ky < 5 and 0 <= kx < 5:
                                t1[pi * 28 + c,
                                   ((dy * 2 + dx) * 2 + mpar) * 12 + u,
                                   ky * 5 + kx] = 1.0
    return t1


def _t2_tap_table():
    tab = -np.ones((72, 16), np.int32)
    for r6 in range(6):
        for j in range(12):
            for dy in range(2):
                for dx in range(2):
                    for u2 in range(4):
                        ky = r6 - dy
                        kx = j - 2 * u2 - dx
                        if 0 <= ky < 5 and 0 <= kx < 5:
                            tab[r6 * 12 + j, (dy * 2 + dx) * 4 + u2] = ky * 5 + kx
    return tab


_T2TAB = _t2_tap_table()


def _w2band_kernel(w2_ref, wl1_ref, out_ref, wl1r_ref):
    zero = jnp.zeros((64, 50), jnp.float32)
    for a in range(72):
        pieces = [w2_ref[int(t)] if t >= 0 else zero for t in _T2TAB[a]]
        out_ref[a * 64:(a + 1) * 64, :] = (
            jnp.concatenate(pieces, axis=1).astype(jnp.bfloat16))
    for m2 in range(4):
        wl1r_ref[:, m2 * 100:(m2 + 1) * 100] = (
            wl1_ref[m2 * 200:(m2 + 1) * 200, :].astype(jnp.bfloat16))


def _build_bands(w2, wl1):
    return pl.pallas_call(
        _w2band_kernel,
        out_shape=(jax.ShapeDtypeStruct((4608, 800), jnp.bfloat16),
                   jax.ShapeDtypeStruct((200, 400), jnp.bfloat16)),
    )(w2, wl1)


_T1 = _build_t1()


def _digit5_kernel(x_ref, w1b_ref, w2b_ref, b2t_ref,
                   wl1_ref, bl1_ref, wl2_ref, bl2_ref, wl3_ref, bl3_ref,
                   out_ref):
    f32 = jnp.float32
    bf16 = jnp.bfloat16
    x = x_ref[...]

    x4 = x.reshape(BI, 7, 4, 28)
    ones = jnp.ones((BI, 6, 2), f32)
    p1 = jnp.concatenate(
        [x4[:, d:d + 6, s, :] for d in range(2) for s in range(4)] + [ones],
        axis=2)
    p1 = p1.reshape(BI * 6, 226).astype(bf16)
    h1 = jnp.dot(p1, w1b_ref[...], preferred_element_type=f32)
    h1 = jnp.maximum(jnp.maximum(h1[:, 0:1536], h1[:, 1536:3072]),
                     jnp.maximum(h1[:, 3072:4608], h1[:, 4608:6144]))
    pooled1 = jnp.maximum(h1, 0.0).astype(bf16)

    xp2 = pooled1.reshape(BI, 6, 1536)
    p2 = jnp.concatenate([xp2[:, q:q + 4, :] for q in range(3)], axis=2)
    p2 = p2.reshape(BI * 4, 4608)
    h2 = jnp.dot(p2, w2b_ref[...], preferred_element_type=f32) + b2t_ref[...]
    h2 = jnp.maximum(jnp.maximum(h2[:, 0:200], h2[:, 200:400]),
                     jnp.maximum(h2[:, 400:600], h2[:, 600:800]))
    feats = jnp.maximum(h2, 0.0).astype(bf16)

    pfc = jnp.dot(feats, wl1_ref[...], preferred_element_type=f32)
    pfc = pfc.reshape(BI, 4, 400)
    h = (pfc[:, 0, 0:100] + pfc[:, 1, 100:200] + pfc[:, 2, 200:300]
         + pfc[:, 3, 300:400] + bl1_ref[...])
    h = jnp.maximum(h, 0.0).astype(bf16)
    h = jnp.dot(h, wl2_ref[...], preferred_element_type=f32) + bl2_ref[...]
    h = jnp.maximum(h, 0.0).astype(bf16)
    z = jnp.dot(h, wl3_ref[...], preferred_element_type=f32) + bl3_ref[...]
    m = jnp.max(z, axis=-1, keepdims=True)
    lse = jnp.log(jnp.sum(jnp.exp(z - m), axis=-1, keepdims=True)) + m
    out_ref[...] = z - lse


def kernel(x, w1, b1, w2, b2, wl1, bl1, wl2, bl2, wl3, bl3, p1, s2, p2):
    B = x.shape[0]
    xp = x.reshape(B, 28, 28)

    w1eff = jnp.sum(w1, axis=1)
    w1band = jnp.einsum("kgt,to->kgo", _T1, w1eff).reshape(224, 6144)
    b1t = jnp.tile(b1, (1, 96))
    b1hi = b1t.astype(jnp.bfloat16).astype(jnp.float32)
    w1band = jnp.concatenate([w1band, b1hi, b1t - b1hi], axis=0)
    w1band = w1band.astype(jnp.bfloat16)
    w2band, wl1r = _build_bands(w2, wl1)
    b2t = jnp.tile(b2, (1, 16))
    wl2 = wl2.astype(jnp.bfloat16)
    wl3 = wl3.astype(jnp.bfloat16)

    in_specs = [
        pl.BlockSpec((BI, 28, 28), lambda b: (b, 0, 0)),
        pl.BlockSpec((226, 6144), lambda b: (0, 0)),
        pl.BlockSpec((4608, 800), lambda b: (0, 0)),
        pl.BlockSpec((1, 800), lambda b: (0, 0)),
        pl.BlockSpec((200, 400), lambda b: (0, 0)),
        pl.BlockSpec((1, 100), lambda b: (0, 0)),
        pl.BlockSpec((100, 100), lambda b: (0, 0)),
        pl.BlockSpec((1, 100), lambda b: (0, 0)),
        pl.BlockSpec((100, 10), lambda b: (0, 0)),
        pl.BlockSpec((1, 10), lambda b: (0, 0)),
    ]
    return pl.pallas_call(
        _digit5_kernel,
        out_shape=jax.ShapeDtypeStruct((B, 10), jnp.float32),
        grid=(B // BI,),
        in_specs=in_specs,
        out_specs=pl.BlockSpec((BI, 10), lambda b: (b, 0)),
        compiler_params=pltpu.CompilerParams(
            dimension_semantics=("parallel",),
            vmem_limit_bytes=60 * 1024 * 1024,
        ),
    )(xp, w1band, w2band, b2t, wl1r, bl1, wl2, bl2, wl3, bl3)


# speedup vs baseline: 1.1693x; 1.0370x over previous
import numpy as np
import jax
import jax.numpy as jnp
from jax.experimental import pallas as pl
from jax.experimental.pallas import tpu as pltpu

BI = 256


def _build_t1():
    t1 = np.zeros((224, 96, 25), np.float32)
    for pi in range(8):
        for c in range(28):
            for dy in range(2):
                for dx in range(2):
                    for mpar in range(2):
                        for u in range(12):
                            ky = pi - 2 * mpar - dy
                            kx = c - 2 * u - dx
                            if 0 <= ky < 5 and 0 <= kx < 5:
                                t1[pi * 28 + c,
                                   ((dy * 2 + dx) * 2 + mpar) * 12 + u,
                                   ky * 5 + kx] = 1.0
    return t1


def _t2_tap_table():
    tab = -np.ones((72, 16), np.int32)
    for r6 in range(6):
        for j in range(12):
            for dy in range(2):
                for dx in range(2):
                    for u2 in range(4):
                        ky = r6 - dy
                        kx = j - 2 * u2 - dx
                        if 0 <= ky < 5 and 0 <= kx < 5:
                            tab[r6 * 12 + j, (dy * 2 + dx) * 4 + u2] = ky * 5 + kx
    return tab


_T2TAB = _t2_tap_table()


def _w2band_kernel(w2_ref, wl1_ref, out_ref, wl1r_ref):
    zero = jnp.zeros((64, 50), jnp.float32)
    for a in range(72):
        pieces = [w2_ref[int(t)] if t >= 0 else zero for t in _T2TAB[a]]
        out_ref[a * 64:(a + 1) * 64, :] = (
            jnp.concatenate(pieces, axis=1).astype(jnp.bfloat16))
    for m2 in range(4):
        wl1r_ref[:, m2 * 100:(m2 + 1) * 100] = (
            wl1_ref[m2 * 200:(m2 + 1) * 200, :].astype(jnp.bfloat16))


def _build_bands(w2, wl1):
    return pl.pallas_call(
        _w2band_kernel,
        out_shape=(jax.ShapeDtypeStruct((4608, 800), jnp.bfloat16),
                   jax.ShapeDtypeStruct((200, 400), jnp.bfloat16)),
    )(w2, wl1)


_T1 = _build_t1()


def _digit5_kernel(x_ref, w1b_ref, w2b_ref, b2t_ref,
                   wl1_ref, bl1_ref, wl2_ref, bl2_ref, wl3_ref, bl3_ref,
                   out_ref):
    f32 = jnp.float32
    bf16 = jnp.bfloat16
    x = x_ref[...]

    x4 = x.reshape(BI, 7, 4, 28)
    ones = jnp.ones((BI, 6, 2), f32)
    p1 = jnp.concatenate(
        [x4[:, d:d + 6, s, :] for d in range(2) for s in range(4)] + [ones],
        axis=2)
    p1 = p1.reshape(BI * 6, 226).astype(bf16)
    h1a = jnp.dot(p1, w1b_ref[:, 0:3072], preferred_element_type=f32)
    pa = jnp.maximum(h1a[:, 0:1536], h1a[:, 1536:3072])
    h1b = jnp.dot(p1, w1b_ref[:, 3072:6144], preferred_element_type=f32)
    pb = jnp.maximum(h1b[:, 0:1536], h1b[:, 1536:3072])
    pooled1 = jnp.maximum(jnp.maximum(pa, pb), 0.0).astype(bf16)

    xp2 = pooled1.reshape(BI, 6, 1536)
    p2 = jnp.concatenate([xp2[:, q:q + 4, :] for q in range(3)], axis=2)
    p2 = p2.reshape(BI * 4, 4608)
    h2 = jnp.dot(p2, w2b_ref[...], preferred_element_type=f32) + b2t_ref[...]
    h2 = jnp.maximum(jnp.maximum(h2[:, 0:200], h2[:, 200:400]),
                     jnp.maximum(h2[:, 400:600], h2[:, 600:800]))
    feats = jnp.maximum(h2, 0.0).astype(bf16)

    pfc = jnp.dot(feats, wl1_ref[...], preferred_element_type=f32)
    pfc = pfc.reshape(BI, 4, 400)
    h = (pfc[:, 0, 0:100] + pfc[:, 1, 100:200] + pfc[:, 2, 200:300]
         + pfc[:, 3, 300:400] + bl1_ref[...])
    h = jnp.maximum(h, 0.0).astype(bf16)
    h = jnp.dot(h, wl2_ref[...], preferred_element_type=f32) + bl2_ref[...]
    h = jnp.maximum(h, 0.0).astype(bf16)
    z = jnp.dot(h, wl3_ref[...], preferred_element_type=f32) + bl3_ref[...]
    m = jnp.max(z, axis=-1, keepdims=True)
    lse = jnp.log(jnp.sum(jnp.exp(z - m), axis=-1, keepdims=True)) + m
    out_ref[...] = z - lse


def kernel(x, w1, b1, w2, b2, wl1, bl1, wl2, bl2, wl3, bl3, p1, s2, p2):
    B = x.shape[0]
    xp = x.reshape(B, 28, 28)

    w1eff = jnp.sum(w1, axis=1)
    w1band = jnp.einsum("kgt,to->kgo", _T1, w1eff).reshape(224, 6144)
    b1t = jnp.tile(b1, (1, 96))
    b1hi = b1t.astype(jnp.bfloat16).astype(jnp.float32)
    w1band = jnp.concatenate([w1band, b1hi, b1t - b1hi], axis=0)
    w1band = w1band.astype(jnp.bfloat16)
    w2band, wl1r = _build_bands(w2, wl1)
    b2t = jnp.tile(b2, (1, 16))
    wl2 = wl2.astype(jnp.bfloat16)
    wl3 = wl3.astype(jnp.bfloat16)

    in_specs = [
        pl.BlockSpec((BI, 28, 28), lambda b: (b, 0, 0)),
        pl.BlockSpec((226, 6144), lambda b: (0, 0)),
        pl.BlockSpec((4608, 800), lambda b: (0, 0)),
        pl.BlockSpec((1, 800), lambda b: (0, 0)),
        pl.BlockSpec((200, 400), lambda b: (0, 0)),
        pl.BlockSpec((1, 100), lambda b: (0, 0)),
        pl.BlockSpec((100, 100), lambda b: (0, 0)),
        pl.BlockSpec((1, 100), lambda b: (0, 0)),
        pl.BlockSpec((100, 10), lambda b: (0, 0)),
        pl.BlockSpec((1, 10), lambda b: (0, 0)),
    ]
    return pl.pallas_call(
        _digit5_kernel,
        out_shape=jax.ShapeDtypeStruct((B, 10), jnp.float32),
        grid=(B // BI,),
        in_specs=in_specs,
        out_specs=pl.BlockSpec((BI, 10), lambda b: (b, 0)),
        compiler_params=pltpu.CompilerParams(
            dimension_semantics=("parallel",),
            vmem_limit_bytes=60 * 1024 * 1024,
        ),
    )(xp, w1band, w2band, b2t, wl1r, bl1, wl2, bl2, wl3, bl3)
